# Initial kernel scaffold; baseline (speedup 1.0000x reference)
import jax
import jax.numpy as jnp
from jax.experimental import pallas as pl


def kernel(distribution, weights, spot_dist):
    raise NotImplementedError("write your pallas kernel here")


# speedup vs baseline: 1.7897x; 1.7897x over previous
import jax
import jax.numpy as jnp
from jax.experimental import pallas as pl


def _body(dist_ref, w_ref, out_ref):
    x = dist_ref[0:1, :]
    y = dist_ref[1:2, :]
    w = w_ref[...]
    s0 = jnp.sum(w, axis=1, keepdims=True)
    s1x = jnp.sum(w * x, axis=1, keepdims=True)
    s1y = jnp.sum(w * y, axis=1, keepdims=True)
    s2x = jnp.sum(w * (x * x), axis=1, keepdims=True)
    s2y = jnp.sum(w * (y * y), axis=1, keepdims=True)
    var_sum = (s2x + s2y) + (s1x * s1x + s1y * s1y) * (s0 - 2.0)
    out_ref[...] = jnp.sum(var_sum, axis=0, keepdims=True) * (0.5 / w.shape[0])


def kernel(distribution, weights, spot_dist):
    del spot_dist
    dist_t = distribution.T
    out = pl.pallas_call(
        _body,
        out_shape=jax.ShapeDtypeStruct((1, 1), jnp.float32),
    )(dist_t, weights)
    return out[0, 0]
